# baseline (speedup 1.0000x reference)
import jax
import jax.numpy as jnp
import numpy as np
from jax.experimental import pallas as pl
from jax.experimental.pallas import tpu as pltpu

N = 512
D = 64
R = 32
RI = 2
BI = 32
NEG = -1e30


def _gcn_block(ct_ref, nei_ref, h_ref, ht_ref,
               c_ref, og_ref,
               relw_ref, relb_ref, ngw_ref, ngb_ref, war_ref, warhjt_ref,
               rels_ref, wars_ref, wnei_ref, wneib_ref,
               hout_ref, cout_ref):
    i = pl.program_id(0)
    bf16 = jnp.bfloat16
    f32 = jnp.float32

    cc = jnp.swapaxes(ct_ref[...].astype(bf16), 1, 2).reshape(BI * N, RI)
    r = jnp.dot(cc, relw_ref[...].astype(bf16), preferred_element_type=f32)
    r = jnp.maximum(r + relb_ref[...], 0.0).astype(bf16)
    ngw = ngw_ref[...].astype(bf16)
    glog = jnp.dot(r, ngw[:R], preferred_element_type=f32)

    h_all = h_ref[...].astype(bf16)
    h_blk = h_ref[pl.ds(i * BI, BI), :]
    a_i = jnp.dot(h_blk.astype(bf16), ngw[R:R + D],
                  preferred_element_type=f32)
    a_i = (a_i + ngb_ref[...]).astype(bf16)
    b_j = jnp.dot(h_all, ngw[R + D:],
                  preferred_element_type=f32).astype(bf16)

    lg = (glog.astype(bf16).reshape(BI, N, D)
          + a_i[:, None, :] + b_j[None, :, :])
    gate = jax.nn.sigmoid(lg)
    q = gate * h_all[None, :, :]

    c0 = ct_ref[:, 0, :]
    c1 = ct_ref[:, 1, :]
    t = jnp.zeros((BI, N), f32)
    for k in range(R):
        rk = jnp.maximum(c0 * rels_ref[0, k] + c1 * rels_ref[1, k]
                         + rels_ref[2, k], 0.0)
        t = t + rk * wars_ref[0, k]
    aw = jnp.dot(h_blk, war_ref[R:R + D], preferred_element_type=f32)
    bw = jnp.dot(warhjt_ref[...], ht_ref[...], preferred_element_type=f32)
    tt = t + aw + bw + wars_ref[1, 0]

    m2 = (nei_ref[...] > 0) & (tt != 0.0)
    mx = jnp.max(jnp.where(m2, tt, NEG), axis=1, keepdims=True)
    w = jnp.where(m2, jnp.exp(tt - mx), 0.0)
    s = jnp.sum(w, axis=1, keepdims=True)
    p = (w / jnp.where(s > 0.0, s, 1.0)).astype(bf16)

    h_sum = jax.lax.dot_general(
        p, q, (((1,), (1,)), ((0,), (0,))),
        preferred_element_type=f32)

    c_out = jnp.dot(h_sum, wnei_ref[...], preferred_element_type=f32)
    c_out = c_out + wneib_ref[...] + c_ref[...]
    cout_ref[...] = c_out
    hout_ref[...] = og_ref[...] * jnp.tanh(c_out)


def kernel(corr_index, nei_index, nei_num, outgate, self_h, self_c,
           rel_W, rel_b, ngate_W, ngate_b, war_W, war_b, wnei_W, wnei_b):
    n = corr_index.shape[0]
    d = self_h.shape[1]
    ri = corr_index.shape[2]
    r = rel_W.shape[1]
    assert (n, d, ri, r) == (N, D, RI, R)
    f32 = jnp.float32

    ct2 = jnp.transpose(corr_index, (0, 2, 1))

    rels = jnp.stack([rel_W[0], rel_W[1], rel_b])
    wars = jnp.zeros((2, r), f32).at[0].set(war_W[:r, 0]).at[1, 0].set(war_b[0])

    grid = (n // BI,)
    full = lambda shape: pl.BlockSpec(shape, lambda i: (0,) * len(shape))
    row_blk = lambda shape: pl.BlockSpec(shape, lambda i: (i,) + (0,) * (len(shape) - 1))
    smem = lambda shape: pl.BlockSpec(shape, lambda i: (0,) * len(shape),
                                      memory_space=pltpu.SMEM)

    h_out, c_out = pl.pallas_call(
        _gcn_block,
        grid=grid,
        in_specs=[
            row_blk((BI, ri, n)),
            row_blk((BI, n)),
            full((n, d)),
            full((d, n)),
            row_blk((BI, d)),
            row_blk((BI, d)),
            full((ri, r)),
            full((1, r)),
            full((r + 2 * d, d)),
            full((1, d)),
            full((r + 2 * d, 1)),
            full((1, d)),
            smem((3, r)),
            smem((2, r)),
            full((d, d)),
            full((1, d)),
        ],
        out_specs=[row_blk((BI, d)), row_blk((BI, d))],
        out_shape=[
            jax.ShapeDtypeStruct((n, d), f32),
            jax.ShapeDtypeStruct((n, d), f32),
        ],
        compiler_params=pltpu.CompilerParams(
            dimension_semantics=("parallel",),
        ),
    )(ct2, nei_index, self_h, self_h.T, self_c, outgate,
      rel_W, rel_b.reshape(1, r), ngate_W, ngate_b.reshape(1, d),
      war_W, war_W[r + d:].T, rels, wars, wnei_W, wnei_b.reshape(1, d))

    return (outgate, h_out, c_out)


# speedup vs baseline: 1.0807x; 1.0807x over previous
import jax
import jax.numpy as jnp
import numpy as np
from jax.experimental import pallas as pl
from jax.experimental.pallas import tpu as pltpu

N = 512
D = 64
R = 32
RI = 2
BI = 32
NEG = -1e30


def _gcn_block(ct_ref, nei_ref, h_ref, ht_ref,
               c_ref, og_ref,
               relw_ref, relb_ref, ngw_ref, ngb_ref, war_ref, warhjt_ref,
               relwt_ref, relbt_ref, warr_ref, wars_ref, wnei_ref, wneib_ref,
               hout_ref, cout_ref):
    i = pl.program_id(0)
    bf16 = jnp.bfloat16
    f32 = jnp.float32

    cc = jnp.swapaxes(ct_ref[...].astype(bf16), 1, 2).reshape(BI * N, RI)
    r = jnp.dot(cc, relw_ref[...].astype(bf16), preferred_element_type=f32)
    r = jnp.maximum(r + relb_ref[...], 0.0).astype(bf16)
    ngw = ngw_ref[...].astype(bf16)
    glog = jnp.dot(r, ngw[:R], preferred_element_type=f32)

    h_all = h_ref[...].astype(bf16)
    h_blk = h_ref[pl.ds(i * BI, BI), :]
    a_i = jnp.dot(h_blk.astype(bf16), ngw[R:R + D],
                  preferred_element_type=f32)
    a_i = (a_i + ngb_ref[...]).astype(bf16)
    b_j = jnp.dot(h_all, ngw[R + D:],
                  preferred_element_type=f32).astype(bf16)

    lg = (glog.astype(bf16).reshape(BI, N, D)
          + a_i[:, None, :] + b_j[None, :, :])
    gate = jax.nn.sigmoid(lg)
    q = gate * h_all[None, :, :]

    c0 = ct_ref[:, 0, :]
    c1 = ct_ref[:, 1, :]
    w0c = relwt_ref[:, 0:1]
    w1c = relwt_ref[:, 1:2]
    rbc = relbt_ref[...]
    wrr = warr_ref[...]
    t_rows = []
    for ii in range(BI):
        riT = jnp.maximum(w0c * c0[ii:ii + 1, :] + w1c * c1[ii:ii + 1, :]
                          + rbc, 0.0)
        t_rows.append(jnp.dot(wrr, riT.astype(bf16),
                              preferred_element_type=f32))
    t = jnp.concatenate(t_rows, axis=0)
    aw = jnp.dot(h_blk, war_ref[R:R + D], preferred_element_type=f32)
    bw = jnp.dot(warhjt_ref[...], ht_ref[...], preferred_element_type=f32)
    tt = t + aw + bw + wars_ref[1, 0]

    m2 = (nei_ref[...] > 0) & (tt != 0.0)
    mx = jnp.max(jnp.where(m2, tt, NEG), axis=1, keepdims=True)
    w = jnp.where(m2, jnp.exp(tt - mx), 0.0)
    s = jnp.sum(w, axis=1, keepdims=True)
    p = (w / jnp.where(s > 0.0, s, 1.0)).astype(bf16)

    h_sum = jax.lax.dot_general(
        p, q, (((1,), (1,)), ((0,), (0,))),
        preferred_element_type=f32)

    c_out = jnp.dot(h_sum, wnei_ref[...], preferred_element_type=f32)
    c_out = c_out + wneib_ref[...] + c_ref[...]
    cout_ref[...] = c_out
    hout_ref[...] = og_ref[...] * jnp.tanh(c_out)


def kernel(corr_index, nei_index, nei_num, outgate, self_h, self_c,
           rel_W, rel_b, ngate_W, ngate_b, war_W, war_b, wnei_W, wnei_b):
    n = corr_index.shape[0]
    d = self_h.shape[1]
    ri = corr_index.shape[2]
    r = rel_W.shape[1]
    assert (n, d, ri, r) == (N, D, RI, R)
    f32 = jnp.float32

    ct2 = jnp.transpose(corr_index, (0, 2, 1))

    rels = jnp.stack([rel_W[0], rel_W[1], rel_b])
    wars = jnp.zeros((2, r), f32).at[0].set(war_W[:r, 0]).at[1, 0].set(war_b[0])

    grid = (n // BI,)
    full = lambda shape: pl.BlockSpec(shape, lambda i: (0,) * len(shape))
    row_blk = lambda shape: pl.BlockSpec(shape, lambda i: (i,) + (0,) * (len(shape) - 1))
    smem = lambda shape: pl.BlockSpec(shape, lambda i: (0,) * len(shape),
                                      memory_space=pltpu.SMEM)

    h_out, c_out = pl.pallas_call(
        _gcn_block,
        grid=grid,
        in_specs=[
            row_blk((BI, ri, n)),
            row_blk((BI, n)),
            full((n, d)),
            full((d, n)),
            row_blk((BI, d)),
            row_blk((BI, d)),
            full((ri, r)),
            full((1, r)),
            full((r + 2 * d, d)),
            full((1, d)),
            full((r + 2 * d, 1)),
            full((1, d)),
            full((r, ri)),
            full((r, 1)),
            full((1, r)),
            smem((2, r)),
            full((d, d)),
            full((1, d)),
        ],
        out_specs=[row_blk((BI, d)), row_blk((BI, d))],
        out_shape=[
            jax.ShapeDtypeStruct((n, d), f32),
            jax.ShapeDtypeStruct((n, d), f32),
        ],
        compiler_params=pltpu.CompilerParams(
            dimension_semantics=("parallel",),
        ),
    )(ct2, nei_index, self_h, self_h.T, self_c, outgate,
      rel_W, rel_b.reshape(1, r), ngate_W, ngate_b.reshape(1, d),
      war_W, war_W[r + d:].T, rel_W.T, rel_b.reshape(r, 1),
      war_W[:r].T.astype(jnp.bfloat16), wars, wnei_W, wnei_b.reshape(1, d))

    return (outgate, h_out, c_out)


# speedup vs baseline: 1.9802x; 1.8324x over previous
import jax
import jax.numpy as jnp
import numpy as np
from jax.experimental import pallas as pl
from jax.experimental.pallas import tpu as pltpu

N = 512
D = 64
R = 32
RI = 2
BI = 32
NEG = -1e30


def _gcn_block(ct_ref, nei_ref, h_ref, ht_ref, c_ref, og_ref,
               relwt_ref, relbt_ref, warr_ref, ngwrt_ref, ngwit_ref,
               ngwjt_ref, ngbt_ref, war_ref, warhjt_ref, warb_ref,
               wnei_ref, wneib_ref,
               hout_ref, cout_ref):
    i = pl.program_id(0)
    bf16 = jnp.bfloat16
    f32 = jnp.float32

    c0 = ct_ref[:, 0, :]
    c1 = ct_ref[:, 1, :]
    w0c = relwt_ref[:, 0:1]
    w1c = relwt_ref[:, 1:2]
    rbc = relbt_ref[...]
    wrr = warr_ref[...]

    h_blk = h_ref[pl.ds(i * BI, BI), :]

    ht = ht_ref[...]
    b_t = jnp.dot(ngwjt_ref[...], ht.astype(bf16),
                  preferred_element_type=f32)
    b_t = (b_t + ngbt_ref[...]).astype(bf16)
    a_t = jnp.dot(ngwit_ref[...], h_blk.astype(bf16).T,
                  preferred_element_type=f32).astype(bf16)
    aw = jnp.dot(h_blk, war_ref[R:R + D], preferred_element_type=f32)
    bw = jnp.dot(warhjt_ref[...], ht, preferred_element_type=f32)

    htb = ht.astype(bf16)
    ngwrt = ngwrt_ref[...]

    t_rows = []
    q_tiles = []
    for ii in range(BI):
        riT = jnp.maximum(w0c * c0[ii:ii + 1, :] + w1c * c1[ii:ii + 1, :]
                          + rbc, 0.0).astype(bf16)
        t_rows.append(jnp.dot(wrr, riT, preferred_element_type=f32))
        glog_t = jnp.dot(ngwrt, riT, preferred_element_type=f32)
        lg_t = glog_t.astype(bf16) + a_t[:, ii:ii + 1] + b_t
        q_tiles.append(jax.nn.sigmoid(lg_t) * htb)
    t = jnp.concatenate(t_rows, axis=0)
    tt = t + aw + bw + warb_ref[...]

    m2 = (nei_ref[...] > 0) & (tt != 0.0)
    mx = jnp.max(jnp.where(m2, tt, NEG), axis=1, keepdims=True)
    w = jnp.where(m2, jnp.exp(tt - mx), 0.0)
    s = jnp.sum(w, axis=1, keepdims=True)
    p = (w / jnp.where(s > 0.0, s, 1.0)).astype(bf16)

    h_cols = [jnp.dot(q_tiles[ii], p[ii:ii + 1, :].T,
                      preferred_element_type=f32) for ii in range(BI)]
    h_sum = jnp.concatenate(h_cols, axis=1).T

    c_out = jnp.dot(h_sum, wnei_ref[...], preferred_element_type=f32)
    c_out = c_out + wneib_ref[...] + c_ref[...]
    cout_ref[...] = c_out
    hout_ref[...] = og_ref[...] * jnp.tanh(c_out)


def kernel(corr_index, nei_index, nei_num, outgate, self_h, self_c,
           rel_W, rel_b, ngate_W, ngate_b, war_W, war_b, wnei_W, wnei_b):
    n = corr_index.shape[0]
    d = self_h.shape[1]
    ri = corr_index.shape[2]
    r = rel_W.shape[1]
    assert (n, d, ri, r) == (N, D, RI, R)
    f32 = jnp.float32
    bf16 = jnp.bfloat16

    ct2 = jnp.transpose(corr_index, (0, 2, 1))

    grid = (n // BI,)
    full = lambda shape: pl.BlockSpec(shape, lambda i: (0,) * len(shape))
    row_blk = lambda shape: pl.BlockSpec(shape, lambda i: (i,) + (0,) * (len(shape) - 1))

    h_out, c_out = pl.pallas_call(
        _gcn_block,
        grid=grid,
        in_specs=[
            row_blk((BI, ri, n)),
            row_blk((BI, n)),
            full((n, d)),
            full((d, n)),
            row_blk((BI, d)),
            row_blk((BI, d)),
            full((r, ri)),
            full((r, 1)),
            full((1, r)),
            full((d, r)),
            full((d, d)),
            full((d, d)),
            full((d, 1)),
            full((r + 2 * d, 1)),
            full((1, d)),
            full((1, 1)),
            full((d, d)),
            full((1, d)),
        ],
        out_specs=[row_blk((BI, d)), row_blk((BI, d))],
        out_shape=[
            jax.ShapeDtypeStruct((n, d), f32),
            jax.ShapeDtypeStruct((n, d), f32),
        ],
        compiler_params=pltpu.CompilerParams(
            dimension_semantics=("parallel",),
        ),
    )(ct2, nei_index, self_h, self_h.T, self_c, outgate,
      rel_W.T, rel_b.reshape(r, 1), war_W[:r].T.astype(bf16),
      ngate_W[:r].T.astype(bf16), ngate_W[r:r + d].T.astype(bf16),
      ngate_W[r + d:].T.astype(bf16), ngate_b.reshape(d, 1),
      war_W, war_W[r + d:].T, war_b.reshape(1, 1),
      wnei_W, wnei_b.reshape(1, d))

    return (outgate, h_out, c_out)


# speedup vs baseline: 2.0988x; 1.0599x over previous
import jax
import jax.numpy as jnp
import numpy as np
from jax.experimental import pallas as pl
from jax.experimental.pallas import tpu as pltpu

N = 512
D = 64
R = 32
RI = 2
BI = 64
NEG = -1e30


def _gcn_block(ct_ref, nei_ref, h_ref, ht_ref, c_ref, og_ref,
               relwt_ref, relbt_ref, warr_ref, ngwrt_ref, ngwit_ref,
               ngwjt_ref, ngbt_ref, war_ref, warhjt_ref, warb_ref,
               wnei_ref, wneib_ref,
               hout_ref, cout_ref):
    i = pl.program_id(0)
    bf16 = jnp.bfloat16
    f32 = jnp.float32

    c0 = ct_ref[:, 0, :]
    c1 = ct_ref[:, 1, :]
    w0c = relwt_ref[:, 0:1]
    w1c = relwt_ref[:, 1:2]
    rbc = relbt_ref[...]
    wrr = warr_ref[...]

    h_blk = h_ref[pl.ds(i * BI, BI), :]

    ht = ht_ref[...]
    b_t = jnp.dot(ngwjt_ref[...], ht.astype(bf16),
                  preferred_element_type=f32)
    b_t = (b_t + ngbt_ref[...]).astype(bf16)
    a_t = jnp.dot(ngwit_ref[...], h_blk.astype(bf16).T,
                  preferred_element_type=f32).astype(bf16)
    aw = jnp.dot(h_blk, war_ref[R:R + D], preferred_element_type=f32)
    bw = jnp.dot(warhjt_ref[...], ht, preferred_element_type=f32)

    htb = ht.astype(bf16)
    ngwrt = ngwrt_ref[...]

    t_rows = []
    q_tiles = []
    for ii in range(BI):
        riT = jnp.maximum(w0c * c0[ii:ii + 1, :] + w1c * c1[ii:ii + 1, :]
                          + rbc, 0.0).astype(bf16)
        t_rows.append(jnp.dot(wrr, riT, preferred_element_type=f32))
        glog_t = jnp.dot(ngwrt, riT, preferred_element_type=f32)
        lg_t = glog_t.astype(bf16) + a_t[:, ii:ii + 1] + b_t
        q_tiles.append(jax.nn.sigmoid(lg_t) * htb)
    t = jnp.concatenate(t_rows, axis=0)
    tt = t + aw + bw + warb_ref[...]

    m2 = (nei_ref[...] > 0) & (tt != 0.0)
    mx = jnp.max(jnp.where(m2, tt, NEG), axis=1, keepdims=True)
    w = jnp.where(m2, jnp.exp(tt - mx), 0.0)
    s = jnp.sum(w, axis=1, keepdims=True)
    p = (w / jnp.where(s > 0.0, s, 1.0)).astype(bf16)

    h_cols = [jnp.dot(q_tiles[ii], p[ii:ii + 1, :].T,
                      preferred_element_type=f32) for ii in range(BI)]
    h_sum = jnp.concatenate(h_cols, axis=1).T

    c_out = jnp.dot(h_sum, wnei_ref[...], preferred_element_type=f32)
    c_out = c_out + wneib_ref[...] + c_ref[...]
    cout_ref[...] = c_out
    hout_ref[...] = og_ref[...] * jnp.tanh(c_out)


def kernel(corr_index, nei_index, nei_num, outgate, self_h, self_c,
           rel_W, rel_b, ngate_W, ngate_b, war_W, war_b, wnei_W, wnei_b):
    n = corr_index.shape[0]
    d = self_h.shape[1]
    ri = corr_index.shape[2]
    r = rel_W.shape[1]
    assert (n, d, ri, r) == (N, D, RI, R)
    f32 = jnp.float32
    bf16 = jnp.bfloat16

    ct2 = jnp.transpose(corr_index, (0, 2, 1))

    grid = (n // BI,)
    full = lambda shape: pl.BlockSpec(shape, lambda i: (0,) * len(shape))
    row_blk = lambda shape: pl.BlockSpec(shape, lambda i: (i,) + (0,) * (len(shape) - 1))

    h_out, c_out = pl.pallas_call(
        _gcn_block,
        grid=grid,
        in_specs=[
            row_blk((BI, ri, n)),
            row_blk((BI, n)),
            full((n, d)),
            full((d, n)),
            row_blk((BI, d)),
            row_blk((BI, d)),
            full((r, ri)),
            full((r, 1)),
            full((1, r)),
            full((d, r)),
            full((d, d)),
            full((d, d)),
            full((d, 1)),
            full((r + 2 * d, 1)),
            full((1, d)),
            full((1, 1)),
            full((d, d)),
            full((1, d)),
        ],
        out_specs=[row_blk((BI, d)), row_blk((BI, d))],
        out_shape=[
            jax.ShapeDtypeStruct((n, d), f32),
            jax.ShapeDtypeStruct((n, d), f32),
        ],
        compiler_params=pltpu.CompilerParams(
            dimension_semantics=("parallel",),
        ),
    )(ct2, nei_index, self_h, self_h.T, self_c, outgate,
      rel_W.T, rel_b.reshape(r, 1), war_W[:r].T.astype(bf16),
      ngate_W[:r].T.astype(bf16), ngate_W[r:r + d].T.astype(bf16),
      ngate_W[r + d:].T.astype(bf16), ngate_b.reshape(d, 1),
      war_W, war_W[r + d:].T, war_b.reshape(1, 1),
      wnei_W, wnei_b.reshape(1, d))

    return (outgate, h_out, c_out)


# speedup vs baseline: 2.3002x; 1.0959x over previous
import jax
import jax.numpy as jnp
import numpy as np
from jax.experimental import pallas as pl
from jax.experimental.pallas import tpu as pltpu

N = 512
D = 64
R = 32
RI = 2
BI = 64
NEG = -1e30


def _gcn_block(ct_ref, nei_ref, h_ref, c_ref, og_ref,
               relw_ref, relb_ref, ngw_ref, ngb_ref, war_ref, warb_ref,
               wnei_ref, wneib_ref,
               hout_ref, cout_ref):
    i = pl.program_id(0)
    bf16 = jnp.bfloat16
    f32 = jnp.float32

    c0 = ct_ref[:, 0, :]
    c1 = ct_ref[:, 1, :]
    relwt = relw_ref[...].T
    w0c = relwt[:, 0:1]
    w1c = relwt[:, 1:2]
    rbc = relb_ref[...].T
    wrr = war_ref[:R].T.astype(bf16)

    h_blk = h_ref[pl.ds(i * BI, BI), :]

    ht = h_ref[...].T
    ngw = ngw_ref[...].astype(bf16)
    b_t = jnp.dot(ngw[R + D:].T, ht.astype(bf16),
                  preferred_element_type=f32)
    b_t = (b_t + ngb_ref[...].T).astype(bf16)
    a_t = jnp.dot(ngw[R:R + D].T, h_blk.astype(bf16).T,
                  preferred_element_type=f32).astype(bf16)
    aw = jnp.dot(h_blk, war_ref[R:R + D], preferred_element_type=f32)
    bw = jnp.dot(war_ref[R + D:].T, ht, preferred_element_type=f32)

    htb = ht.astype(bf16)
    ngwrt = ngw[:R].T

    t_rows = []
    q_tiles = []
    for ii in range(BI):
        riT = jnp.maximum(w0c * c0[ii:ii + 1, :] + w1c * c1[ii:ii + 1, :]
                          + rbc, 0.0).astype(bf16)
        t_rows.append(jnp.dot(wrr, riT, preferred_element_type=f32))
        glog_t = jnp.dot(ngwrt, riT, preferred_element_type=f32)
        lg_t = glog_t.astype(bf16) + a_t[:, ii:ii + 1] + b_t
        q_tiles.append(jax.nn.sigmoid(lg_t) * htb)
    t = jnp.concatenate(t_rows, axis=0)
    tt = t + aw + bw + warb_ref[...]

    m2 = (nei_ref[...] > 0) & (tt != 0.0)
    mx = jnp.max(jnp.where(m2, tt, NEG), axis=1, keepdims=True)
    w = jnp.where(m2, jnp.exp(tt - mx), 0.0)
    s = jnp.sum(w, axis=1, keepdims=True)
    p = (w / jnp.where(s > 0.0, s, 1.0)).astype(bf16)

    h_cols = [jnp.dot(q_tiles[ii], p[ii:ii + 1, :].T,
                      preferred_element_type=f32) for ii in range(BI)]
    h_sum = jnp.concatenate(h_cols, axis=1).T

    c_out = jnp.dot(h_sum, wnei_ref[...], preferred_element_type=f32)
    c_out = c_out + wneib_ref[...] + c_ref[...]
    cout_ref[...] = c_out
    hout_ref[...] = og_ref[...] * jnp.tanh(c_out)


def kernel(corr_index, nei_index, nei_num, outgate, self_h, self_c,
           rel_W, rel_b, ngate_W, ngate_b, war_W, war_b, wnei_W, wnei_b):
    n = corr_index.shape[0]
    d = self_h.shape[1]
    ri = corr_index.shape[2]
    r = rel_W.shape[1]
    assert (n, d, ri, r) == (N, D, RI, R)
    f32 = jnp.float32
    bf16 = jnp.bfloat16

    ct2 = jnp.transpose(corr_index, (0, 2, 1))

    grid = (n // BI,)
    full = lambda shape: pl.BlockSpec(shape, lambda i: (0,) * len(shape))
    row_blk = lambda shape: pl.BlockSpec(shape, lambda i: (i,) + (0,) * (len(shape) - 1))

    h_out, c_out = pl.pallas_call(
        _gcn_block,
        grid=grid,
        in_specs=[
            row_blk((BI, ri, n)),
            row_blk((BI, n)),
            full((n, d)),
            row_blk((BI, d)),
            row_blk((BI, d)),
            full((ri, r)),
            full((1, r)),
            full((r + 2 * d, d)),
            full((1, d)),
            full((r + 2 * d, 1)),
            full((1, 1)),
            full((d, d)),
            full((1, d)),
        ],
        out_specs=[row_blk((BI, d)), row_blk((BI, d))],
        out_shape=[
            jax.ShapeDtypeStruct((n, d), f32),
            jax.ShapeDtypeStruct((n, d), f32),
        ],
        compiler_params=pltpu.CompilerParams(
            dimension_semantics=("parallel",),
        ),
    )(ct2, nei_index, self_h, self_c, outgate,
      rel_W, rel_b.reshape(1, r), ngate_W, ngate_b.reshape(1, d),
      war_W, war_b.reshape(1, 1), wnei_W, wnei_b.reshape(1, d))

    return (outgate, h_out, c_out)


# speedup vs baseline: 2.4774x; 1.0770x over previous
import jax
import jax.numpy as jnp
import numpy as np
from jax.experimental import pallas as pl
from jax.experimental.pallas import tpu as pltpu

N = 512
D = 64
R = 32
RI = 2
BI = 128
NEG = -1e30


def _gcn_block(ct_ref, nei_ref, h_ref, c_ref, og_ref,
               relw_ref, relb_ref, ngw_ref, ngb_ref, war_ref, warb_ref,
               wnei_ref, wneib_ref,
               hout_ref, cout_ref):
    i = pl.program_id(0)
    bf16 = jnp.bfloat16
    f32 = jnp.float32

    c0 = ct_ref[:, 0, :]
    c1 = ct_ref[:, 1, :]
    relwt = relw_ref[...].T
    w0c = relwt[:, 0:1]
    w1c = relwt[:, 1:2]
    rbc = relb_ref[...].T
    wrr = war_ref[:R].T.astype(bf16)

    h_blk = h_ref[pl.ds(i * BI, BI), :]

    ht = h_ref[...].T
    ngw = ngw_ref[...].astype(bf16)
    b_t = jnp.dot(ngw[R + D:].T, ht.astype(bf16),
                  preferred_element_type=f32)
    b_t = (b_t + ngb_ref[...].T).astype(bf16)
    a_t = jnp.dot(ngw[R:R + D].T, h_blk.astype(bf16).T,
                  preferred_element_type=f32).astype(bf16)
    aw = jnp.dot(h_blk, war_ref[R:R + D], preferred_element_type=f32)
    bw = jnp.dot(war_ref[R + D:].T, ht, preferred_element_type=f32)

    htb = ht.astype(bf16)
    ngwrt = ngw[:R].T

    t_rows = []
    q_tiles = []
    for ii in range(BI):
        riT = jnp.maximum(w0c * c0[ii:ii + 1, :] + w1c * c1[ii:ii + 1, :]
                          + rbc, 0.0).astype(bf16)
        t_rows.append(jnp.dot(wrr, riT, preferred_element_type=f32))
        glog_t = jnp.dot(ngwrt, riT, preferred_element_type=f32)
        lg_t = glog_t.astype(bf16) + a_t[:, ii:ii + 1] + b_t
        q_tiles.append(jax.nn.sigmoid(lg_t) * htb)
    t = jnp.concatenate(t_rows, axis=0)
    tt = t + aw + bw + warb_ref[...]

    m2 = (nei_ref[...] > 0) & (tt != 0.0)
    mx = jnp.max(jnp.where(m2, tt, NEG), axis=1, keepdims=True)
    w = jnp.where(m2, jnp.exp(tt - mx), 0.0)
    s = jnp.sum(w, axis=1, keepdims=True)
    p = (w / jnp.where(s > 0.0, s, 1.0)).astype(bf16)

    h_cols = [jnp.dot(q_tiles[ii], p[ii:ii + 1, :].T,
                      preferred_element_type=f32) for ii in range(BI)]
    h_sum = jnp.concatenate(h_cols, axis=1).T

    c_out = jnp.dot(h_sum, wnei_ref[...], preferred_element_type=f32)
    c_out = c_out + wneib_ref[...] + c_ref[...]
    cout_ref[...] = c_out
    hout_ref[...] = og_ref[...] * jnp.tanh(c_out)


def kernel(corr_index, nei_index, nei_num, outgate, self_h, self_c,
           rel_W, rel_b, ngate_W, ngate_b, war_W, war_b, wnei_W, wnei_b):
    n = corr_index.shape[0]
    d = self_h.shape[1]
    ri = corr_index.shape[2]
    r = rel_W.shape[1]
    assert (n, d, ri, r) == (N, D, RI, R)
    f32 = jnp.float32
    bf16 = jnp.bfloat16

    ct2 = jnp.transpose(corr_index, (0, 2, 1))

    grid = (n // BI,)
    full = lambda shape: pl.BlockSpec(shape, lambda i: (0,) * len(shape))
    row_blk = lambda shape: pl.BlockSpec(shape, lambda i: (i,) + (0,) * (len(shape) - 1))

    h_out, c_out = pl.pallas_call(
        _gcn_block,
        grid=grid,
        in_specs=[
            row_blk((BI, ri, n)),
            row_blk((BI, n)),
            full((n, d)),
            row_blk((BI, d)),
            row_blk((BI, d)),
            full((ri, r)),
            full((1, r)),
            full((r + 2 * d, d)),
            full((1, d)),
            full((r + 2 * d, 1)),
            full((1, 1)),
            full((d, d)),
            full((1, d)),
        ],
        out_specs=[row_blk((BI, d)), row_blk((BI, d))],
        out_shape=[
            jax.ShapeDtypeStruct((n, d), f32),
            jax.ShapeDtypeStruct((n, d), f32),
        ],
        compiler_params=pltpu.CompilerParams(
            dimension_semantics=("parallel",),
        ),
    )(ct2, nei_index, self_h, self_c, outgate,
      rel_W, rel_b.reshape(1, r), ngate_W, ngate_b.reshape(1, d),
      war_W, war_b.reshape(1, 1), wnei_W, wnei_b.reshape(1, d))

    return (outgate, h_out, c_out)
